# baseline (speedup 1.0000x reference)
import functools

import jax
import jax.numpy as jnp
from jax import lax
from jax.experimental import pallas as pl
from jax.experimental.pallas import tpu as pltpu
from jax.experimental.pallas import tpu_sc as plsc

N = 819200
NC, NS, L = 2, 16, 16
NW = NC * NS
PER_W = N // NW
CHUNK = 1280
CR = CHUNK // 128
NCHUNK = PER_W // CHUNK
D = 16
N_TBL = 1000000


def _sc_distance(xf, yf, table):
    mesh = plsc.VectorSubcoreMesh(
        core_axis_name="c", subcore_axis_name="s", num_cores=NC, num_subcores=NS
    )

    @functools.partial(
        pl.kernel,
        out_type=jax.ShapeDtypeStruct((N,), jnp.float32),
        mesh=mesh,
        compiler_params=pltpu.CompilerParams(
            needs_layout_passes=False, use_tc_tiling_on_sc=False),
        scratch_types=[
            pltpu.VMEM((CHUNK,), jnp.int32),
            pltpu.VMEM((CHUNK,), jnp.int32),
            pltpu.VMEM((CHUNK,), jnp.int32),
            pltpu.VMEM((CHUNK,), jnp.int32),
            pltpu.VMEM((CHUNK, D), jnp.float32),
            pltpu.VMEM((CHUNK, D), jnp.float32),
            pltpu.VMEM((CHUNK, D), jnp.float32),
            pltpu.VMEM((CHUNK, D), jnp.float32),
            pltpu.VMEM((CHUNK,), jnp.float32),
            pltpu.VMEM((CHUNK,), jnp.float32),
            pltpu.SemaphoreType.DMA,
            pltpu.SemaphoreType.DMA,
        ],
    )
    def k(x_hbm, y_hbm, tbl_hbm, out_hbm,
          ix0, iy0, ix1, iy1, rx0, ry0, rx1, ry1, do0, do1, sem0, sem1):
        wid = lax.axis_index("s") * NC + lax.axis_index("c")

        def stage_fire(c, ix, iy, rx, ry, sem):
            e0 = wid * PER_W + c * CHUNK
            pltpu.sync_copy(x_hbm.at[pl.ds(e0, CHUNK)], ix)
            pltpu.sync_copy(y_hbm.at[pl.ds(e0, CHUNK)], iy)
            for j in range(CR):
                pltpu.async_copy(
                    tbl_hbm.at[ix.at[pl.ds(j * 128, 128)]],
                    rx.at[pl.ds(j * 128, 128)], sem)
                pltpu.async_copy(
                    tbl_hbm.at[iy.at[pl.ds(j * 128, 128)]],
                    ry.at[pl.ds(j * 128, 128)], sem)

        def drain(rx, ry, sem):
            pltpu.make_async_copy(tbl_hbm.at[pl.ds(0, CHUNK)], rx, sem).wait()
            pltpu.make_async_copy(tbl_hbm.at[pl.ds(0, CHUNK)], ry, sem).wait()

        def compute(c, rx, ry, do):
            e0 = wid * PER_W + c * CHUNK

            @plsc.parallel_loop(0, CHUNK // L, unroll=4)
            def group_body(g):
                ge = g * L
                eidx = ge + lax.iota(jnp.int32, L)
                uu = jnp.zeros((L,), jnp.float32)
                vv = jnp.zeros((L,), jnp.float32)
                uv = jnp.zeros((L,), jnp.float32)
                for dd in range(D):
                    dvec = jnp.full((L,), dd, jnp.int32)
                    cu = plsc.load_gather(rx, [eidx, dvec])
                    cv = plsc.load_gather(ry, [eidx, dvec])
                    uu = uu + cu * cu
                    vv = vv + cv * cv
                    uv = uv + cu * cv
                ww = jnp.maximum(uu + vv - 2.0 * uv, 0.0)
                den = (1.0 - uu) * (1.0 - vv)
                do[pl.ds(ge, L)] = 1.0 + 2.0 * ww / den
            pltpu.sync_copy(do, out_hbm.at[pl.ds(e0, CHUNK)])

        with jax.named_scope("prologue_fire"):
            stage_fire(0, ix0, iy0, rx0, ry0, sem0)

        def body(i, carry):
            c0 = 2 * i
            stage_fire(c0 + 1, ix1, iy1, rx1, ry1, sem1)
            with jax.named_scope("drain0"):
                drain(rx0, ry0, sem0)
            with jax.named_scope("compute0"):
                compute(c0, rx0, ry0, do0)
            stage_fire(c0 + 2, ix0, iy0, rx0, ry0, sem0)
            with jax.named_scope("drain1"):
                drain(rx1, ry1, sem1)
            with jax.named_scope("compute1"):
                compute(c0 + 1, rx1, ry1, do1)
            return carry

        lax.fori_loop(0, NCHUNK // 2 - 1, body, 0)

        c0 = NCHUNK - 2
        stage_fire(c0 + 1, ix1, iy1, rx1, ry1, sem1)
        drain(rx0, ry0, sem0)
        compute(c0, rx0, ry0, do0)
        drain(rx1, ry1, sem1)
        compute(c0 + 1, rx1, ry1, do1)

    return k(xf, yf, table)


def _acosh_body(d_ref, o_ref):
    d = d_ref[...]
    o_ref[...] = -jnp.log(d + jnp.sqrt((d - 1.0) * (d + 1.0)))


def _tc_neg_acosh(d):
    d2 = d.reshape(N // 128, 128)
    out = pl.pallas_call(
        _acosh_body,
        out_shape=jax.ShapeDtypeStruct((N // 128, 128), jnp.float32),
    )(d2)
    return out


def kernel(x, y, embedding):
    xf = x.reshape(N)
    yf = y.reshape(N)
    d = _sc_distance(xf, yf, embedding)
    return _tc_neg_acosh(d).reshape(x.shape)


# speedup vs baseline: 1.8131x; 1.8131x over previous
import functools

import jax
import jax.numpy as jnp
from jax import lax
from jax.experimental import pallas as pl
from jax.experimental.pallas import tpu as pltpu
from jax.experimental.pallas import tpu_sc as plsc

N = 819200
NC, NS, L = 2, 16, 16
NW = NC * NS
PER_W = N // NW
CHUNK = 1280
CR = CHUNK // 128
NCHUNK = PER_W // CHUNK
D = 16
N_TBL = 1000000


def _sc_distance(xf, yf, table):
    mesh = plsc.VectorSubcoreMesh(
        core_axis_name="c", subcore_axis_name="s", num_cores=NC, num_subcores=NS
    )

    @functools.partial(
        pl.kernel,
        out_type=jax.ShapeDtypeStruct((N,), jnp.float32),
        mesh=mesh,
        compiler_params=pltpu.CompilerParams(
            needs_layout_passes=False, use_tc_tiling_on_sc=False),
        scratch_types=[
            pltpu.VMEM((CHUNK,), jnp.int32),
            pltpu.VMEM((CHUNK,), jnp.int32),
            pltpu.VMEM((CHUNK,), jnp.int32),
            pltpu.VMEM((CHUNK,), jnp.int32),
            pltpu.VMEM((CHUNK, D), jnp.float32),
            pltpu.VMEM((CHUNK, D), jnp.float32),
            pltpu.VMEM((CHUNK, D), jnp.float32),
            pltpu.VMEM((CHUNK, D), jnp.float32),
            pltpu.VMEM((CHUNK,), jnp.float32),
            pltpu.VMEM((CHUNK,), jnp.float32),
            pltpu.SemaphoreType.DMA,
            pltpu.SemaphoreType.DMA,
        ],
    )
    def k(x_hbm, y_hbm, tbl_hbm, out_hbm,
          ix0, iy0, ix1, iy1, rx0, ry0, rx1, ry1, do0, do1, sem0, sem1):
        wid = lax.axis_index("s") * NC + lax.axis_index("c")

        def stage_fire(c, ix, iy, rx, ry, sem):
            e0 = wid * PER_W + c * CHUNK
            pltpu.sync_copy(x_hbm.at[pl.ds(e0, CHUNK)], ix)
            pltpu.sync_copy(y_hbm.at[pl.ds(e0, CHUNK)], iy)
            for j in range(CR):
                pltpu.async_copy(
                    tbl_hbm.at[ix.at[pl.ds(j * 128, 128)]],
                    rx.at[pl.ds(j * 128, 128)], sem)
                pltpu.async_copy(
                    tbl_hbm.at[iy.at[pl.ds(j * 128, 128)]],
                    ry.at[pl.ds(j * 128, 128)], sem)

        def drain(rx, ry, sem):
            pltpu.make_async_copy(tbl_hbm.at[pl.ds(0, CHUNK)], rx, sem).wait()
            pltpu.make_async_copy(tbl_hbm.at[pl.ds(0, CHUNK)], ry, sem).wait()

        def compute(c, rx, ry, do):
            e0 = wid * PER_W + c * CHUNK

            def group_body(g, carry2):
                ge = g * L
                eidx = ge + lax.iota(jnp.int32, L)
                uu = jnp.zeros((L,), jnp.float32)
                vv = jnp.zeros((L,), jnp.float32)
                uv = jnp.zeros((L,), jnp.float32)
                for dd in range(D):
                    dvec = jnp.full((L,), dd, jnp.int32)
                    cu = plsc.load_gather(rx, [eidx, dvec])
                    cv = plsc.load_gather(ry, [eidx, dvec])
                    uu = uu + cu * cu
                    vv = vv + cv * cv
                    uv = uv + cu * cv
                ww = jnp.maximum(uu + vv - 2.0 * uv, 0.0)
                den = (1.0 - uu) * (1.0 - vv)
                do[pl.ds(ge, L)] = 1.0 + 2.0 * ww / den
                return carry2

            lax.fori_loop(0, CHUNK // L, group_body, 0)
            pltpu.sync_copy(do, out_hbm.at[pl.ds(e0, CHUNK)])

        with jax.named_scope("prologue_fire"):
            stage_fire(0, ix0, iy0, rx0, ry0, sem0)

        def body(i, carry):
            c0 = 2 * i
            stage_fire(c0 + 1, ix1, iy1, rx1, ry1, sem1)
            with jax.named_scope("drain0"):
                drain(rx0, ry0, sem0)
            with jax.named_scope("compute0"):
                compute(c0, rx0, ry0, do0)
            stage_fire(c0 + 2, ix0, iy0, rx0, ry0, sem0)
            with jax.named_scope("drain1"):
                drain(rx1, ry1, sem1)
            with jax.named_scope("compute1"):
                compute(c0 + 1, rx1, ry1, do1)
            return carry

        lax.fori_loop(0, NCHUNK // 2 - 1, body, 0)

        c0 = NCHUNK - 2
        stage_fire(c0 + 1, ix1, iy1, rx1, ry1, sem1)
        drain(rx0, ry0, sem0)
        compute(c0, rx0, ry0, do0)
        drain(rx1, ry1, sem1)
        compute(c0 + 1, rx1, ry1, do1)

    return k(xf, yf, table)


NT_FULL = N_TBL // 128
NT_PER_W = NT_FULL // NW
NT_EXTRA = NT_FULL - NW * NT_PER_W
TAIL_COLS = N_TBL - NT_FULL * 128


def _sc_pack_table(embT):
    mesh = plsc.VectorSubcoreMesh(
        core_axis_name="c", subcore_axis_name="s", num_cores=NC, num_subcores=NS
    )

    @functools.partial(
        pl.kernel,
        out_type=jax.ShapeDtypeStruct((N_TBL // 8, 128), jnp.float32),
        mesh=mesh,
        compiler_params=pltpu.CompilerParams(
            needs_layout_passes=False, use_tc_tiling_on_sc=True),
        scratch_types=(
            [pltpu.VMEM((8, 128), jnp.float32) for _ in range(8)]
            + [pltpu.VMEM((16, 128), jnp.float32) for _ in range(4)]
            + [pltpu.SemaphoreType.DMA for _ in range(8)]
        ),
    )
    def k(t_hbm, out_hbm, a0, b0, a1, b1, a2, b2, a3, b3,
          o0, o1, o2, o3, sr0, sr1, sr2, sr3, sw0, sw1, sw2, sw3):
        ci = lax.axis_index("c")
        si = lax.axis_index("s")
        wid = si * NC + ci
        abufs = (a0, a1, a2, a3)
        bbufs = (b0, b1, b2, b3)
        obufs = (o0, o1, o2, o3)
        srs = (sr0, sr1, sr2, sr3)
        sws = (sw0, sw1, sw2, sw3)
        t0w = wid * NT_PER_W

        def fire_read(t, b):
            c0 = t * 128
            pltpu.async_copy(
                t_hbm.at[pl.ds(0, 8), pl.ds(c0, 128)], abufs[b], srs[b])
            pltpu.async_copy(
                t_hbm.at[pl.ds(8, 8), pl.ds(c0, 128)], bbufs[b], srs[b])

        def wait_read(b):
            pltpu.make_async_copy(
                t_hbm.at[pl.ds(0, 8), pl.ds(0, 128)], abufs[b], srs[b]).wait()
            pltpu.make_async_copy(
                t_hbm.at[pl.ds(0, 8), pl.ds(0, 128)], bbufs[b], srs[b]).wait()

        def fire_owrite(t, b):
            pltpu.async_copy(
                obufs[b], out_hbm.at[pl.ds(t * 16, 16)], sws[b])

        def wait_owrite(b):
            pltpu.make_async_copy(
                obufs[b], out_hbm.at[pl.ds(0, 16)], sws[b]).wait()

        def transpose_tile(b):
            ost = obufs[b]
            for g in range(8):
                cols = g * 16 + lax.iota(jnp.int32, L)
                qq = cols >> 3
                cbase = (cols & 7) << 4
                for d in range(8):
                    va = abufs[b][d, pl.ds(g * 16, 16)]
                    plsc.store_scatter(ost, [qq, cbase + d], va)
                    vb = bbufs[b][d, pl.ds(g * 16, 16)]
                    plsc.store_scatter(ost, [qq, cbase + (d + 8)], vb)

        for b in range(4):
            fire_read(t0w + b, b)
        for b in range(4):
            wait_read(b)
            transpose_tile(b)
            fire_owrite(t0w + b, b)
            fire_read(t0w + b + 4, b)

        def ring_body(j, carry):
            t = t0w + 4 * j
            for b in range(4):
                wait_read(b)
                wait_owrite(b)
                transpose_tile(b)
                fire_owrite(t + b, b)
                fire_read(t + b + 4, b)
            return carry

        lax.fori_loop(1, NT_PER_W // 4 - 1, ring_body, 0)

        t = t0w + NT_PER_W - 4
        for b in range(4):
            wait_read(b)
            wait_owrite(b)
            transpose_tile(b)
            fire_owrite(t + b, b)
        for b in range(4):
            wait_owrite(b)

        @pl.when(wid < NT_EXTRA)
        def _extra():
            te = NW * NT_PER_W + wid
            fire_read(te, 0)
            wait_read(0)
            transpose_tile(0)
            fire_owrite(te, 0)
            wait_owrite(0)

    return k(embT)


def _acosh_body(d_ref, o_ref):
    d = d_ref[...]
    o_ref[...] = -jnp.log(d + jnp.sqrt((d - 1.0) * (d + 1.0)))


def _tc_neg_acosh(d):
    d2 = d.reshape(N // 128, 128)
    out = pl.pallas_call(
        _acosh_body,
        out_shape=jax.ShapeDtypeStruct((N // 128, 128), jnp.float32),
    )(d2)
    return out


def kernel(x, y, embedding):
    xf = x.reshape(N)
    yf = y.reshape(N)
    t128 = _sc_pack_table(embedding.T)
    tail = embedding[NT_FULL * 128:, :].reshape(TAIL_COLS // 8, 128)
    t128 = lax.dynamic_update_slice(t128, tail, (NT_FULL * 16, 0))
    d = _sc_distance(xf, yf, t128.reshape(N_TBL, D))
    return _tc_neg_acosh(d).reshape(x.shape)


# speedup vs baseline: 2.1924x; 1.2092x over previous
import functools

import jax
import jax.numpy as jnp
from jax import lax
from jax.experimental import pallas as pl
from jax.experimental.pallas import tpu as pltpu
from jax.experimental.pallas import tpu_sc as plsc

N = 819200
NC, NS, L = 2, 16, 16
NW = NC * NS
PER_W = N // NW
CHUNK = 1280
CR = CHUNK // 128
NCHUNK = PER_W // CHUNK
D = 16
N_TBL = 1000000


def _sc_distance(xf, yf, table):
    mesh = plsc.VectorSubcoreMesh(
        core_axis_name="c", subcore_axis_name="s", num_cores=NC, num_subcores=NS
    )

    @functools.partial(
        pl.kernel,
        out_type=jax.ShapeDtypeStruct((N,), jnp.float32),
        mesh=mesh,
        compiler_params=pltpu.CompilerParams(
            needs_layout_passes=False, use_tc_tiling_on_sc=False),
        scratch_types=[
            pltpu.VMEM((CHUNK,), jnp.int32),
            pltpu.VMEM((CHUNK,), jnp.int32),
            pltpu.VMEM((CHUNK,), jnp.int32),
            pltpu.VMEM((CHUNK,), jnp.int32),
            pltpu.VMEM((CHUNK, D), jnp.float32),
            pltpu.VMEM((CHUNK, D), jnp.float32),
            pltpu.VMEM((CHUNK, D), jnp.float32),
            pltpu.VMEM((CHUNK, D), jnp.float32),
            pltpu.VMEM((CHUNK,), jnp.float32),
            pltpu.VMEM((CHUNK,), jnp.float32),
            pltpu.SemaphoreType.DMA,
            pltpu.SemaphoreType.DMA,
        ],
    )
    def k(x_hbm, y_hbm, tbl_hbm, out_hbm,
          ix0, iy0, ix1, iy1, rx0, ry0, rx1, ry1, do0, do1, sem0, sem1):
        wid = lax.axis_index("s") * NC + lax.axis_index("c")

        def stage_fire(c, ix, iy, rx, ry, sem):
            e0 = wid * PER_W + c * CHUNK
            pltpu.sync_copy(x_hbm.at[pl.ds(e0, CHUNK)], ix)
            pltpu.sync_copy(y_hbm.at[pl.ds(e0, CHUNK)], iy)
            for j in range(CR):
                pltpu.async_copy(
                    tbl_hbm.at[ix.at[pl.ds(j * 128, 128)]],
                    rx.at[pl.ds(j * 128, 128)], sem)
                pltpu.async_copy(
                    tbl_hbm.at[iy.at[pl.ds(j * 128, 128)]],
                    ry.at[pl.ds(j * 128, 128)], sem)

        def drain(rx, ry, sem):
            pltpu.make_async_copy(tbl_hbm.at[pl.ds(0, CHUNK)], rx, sem).wait()
            pltpu.make_async_copy(tbl_hbm.at[pl.ds(0, CHUNK)], ry, sem).wait()

        def compute(c, rx, ry, do):
            e0 = wid * PER_W + c * CHUNK

            def group_body(g, carry2):
                ge = g * L
                eidx = ge + lax.iota(jnp.int32, L)
                acc = [[jnp.zeros((L,), jnp.float32) for _ in range(3)]
                       for _ in range(4)]
                for dd in range(D):
                    dvec = jnp.full((L,), dd, jnp.int32)
                    cu = plsc.load_gather(rx, [eidx, dvec])
                    cv = plsc.load_gather(ry, [eidx, dvec])
                    a = acc[dd % 4]
                    a[0] = a[0] + cu * cu
                    a[1] = a[1] + cv * cv
                    a[2] = a[2] + cu * cv
                uu = (acc[0][0] + acc[1][0]) + (acc[2][0] + acc[3][0])
                vv = (acc[0][1] + acc[1][1]) + (acc[2][1] + acc[3][1])
                uv = (acc[0][2] + acc[1][2]) + (acc[2][2] + acc[3][2])
                ww = jnp.maximum(uu + vv - 2.0 * uv, 0.0)
                den = (1.0 - uu) * (1.0 - vv)
                do[pl.ds(ge, L)] = 1.0 + 2.0 * ww / den
                return carry2

            lax.fori_loop(0, CHUNK // L, group_body, 0)
            pltpu.sync_copy(do, out_hbm.at[pl.ds(e0, CHUNK)])

        with jax.named_scope("prologue_fire"):
            stage_fire(0, ix0, iy0, rx0, ry0, sem0)

        def body(i, carry):
            c0 = 2 * i
            stage_fire(c0 + 1, ix1, iy1, rx1, ry1, sem1)
            with jax.named_scope("drain0"):
                drain(rx0, ry0, sem0)
            with jax.named_scope("compute0"):
                compute(c0, rx0, ry0, do0)
            stage_fire(c0 + 2, ix0, iy0, rx0, ry0, sem0)
            with jax.named_scope("drain1"):
                drain(rx1, ry1, sem1)
            with jax.named_scope("compute1"):
                compute(c0 + 1, rx1, ry1, do1)
            return carry

        lax.fori_loop(0, NCHUNK // 2 - 1, body, 0)

        c0 = NCHUNK - 2
        stage_fire(c0 + 1, ix1, iy1, rx1, ry1, sem1)
        drain(rx0, ry0, sem0)
        compute(c0, rx0, ry0, do0)
        drain(rx1, ry1, sem1)
        compute(c0 + 1, rx1, ry1, do1)

    return k(xf, yf, table)


NT_FULL = N_TBL // 128
NT_PER_W = NT_FULL // NW
NT_EXTRA = NT_FULL - NW * NT_PER_W
TAIL_COLS = N_TBL - NT_FULL * 128


def _sc_pack_table(embT):
    mesh = plsc.VectorSubcoreMesh(
        core_axis_name="c", subcore_axis_name="s", num_cores=NC, num_subcores=NS
    )

    @functools.partial(
        pl.kernel,
        out_type=jax.ShapeDtypeStruct((N_TBL // 8, 128), jnp.float32),
        mesh=mesh,
        compiler_params=pltpu.CompilerParams(
            needs_layout_passes=False, use_tc_tiling_on_sc=True),
        scratch_types=(
            [pltpu.VMEM((8, 128), jnp.float32) for _ in range(8)]
            + [pltpu.VMEM((16, 128), jnp.float32) for _ in range(4)]
            + [pltpu.SemaphoreType.DMA for _ in range(8)]
        ),
    )
    def k(t_hbm, out_hbm, a0, b0, a1, b1, a2, b2, a3, b3,
          o0, o1, o2, o3, sr0, sr1, sr2, sr3, sw0, sw1, sw2, sw3):
        ci = lax.axis_index("c")
        si = lax.axis_index("s")
        wid = si * NC + ci
        abufs = (a0, a1, a2, a3)
        bbufs = (b0, b1, b2, b3)
        obufs = (o0, o1, o2, o3)
        srs = (sr0, sr1, sr2, sr3)
        sws = (sw0, sw1, sw2, sw3)
        t0w = wid * NT_PER_W

        def fire_read(t, b):
            c0 = t * 128
            pltpu.async_copy(
                t_hbm.at[pl.ds(0, 8), pl.ds(c0, 128)], abufs[b], srs[b])
            pltpu.async_copy(
                t_hbm.at[pl.ds(8, 8), pl.ds(c0, 128)], bbufs[b], srs[b])

        def wait_read(b):
            pltpu.make_async_copy(
                t_hbm.at[pl.ds(0, 8), pl.ds(0, 128)], abufs[b], srs[b]).wait()
            pltpu.make_async_copy(
                t_hbm.at[pl.ds(0, 8), pl.ds(0, 128)], bbufs[b], srs[b]).wait()

        def fire_owrite(t, b):
            pltpu.async_copy(
                obufs[b], out_hbm.at[pl.ds(t * 16, 16)], sws[b])

        def wait_owrite(b):
            pltpu.make_async_copy(
                obufs[b], out_hbm.at[pl.ds(0, 16)], sws[b]).wait()

        def transpose_tile(b):
            ost = obufs[b]
            for g in range(8):
                cols = g * 16 + lax.iota(jnp.int32, L)
                qq = cols >> 3
                cbase = (cols & 7) << 4
                for d in range(8):
                    va = abufs[b][d, pl.ds(g * 16, 16)]
                    plsc.store_scatter(ost, [qq, cbase + d], va)
                    vb = bbufs[b][d, pl.ds(g * 16, 16)]
                    plsc.store_scatter(ost, [qq, cbase + (d + 8)], vb)

        for b in range(4):
            fire_read(t0w + b, b)
        for b in range(4):
            wait_read(b)
            transpose_tile(b)
            fire_owrite(t0w + b, b)
            fire_read(t0w + b + 4, b)

        def ring_body(j, carry):
            t = t0w + 4 * j
            for b in range(4):
                wait_read(b)
                wait_owrite(b)
                transpose_tile(b)
                fire_owrite(t + b, b)
                fire_read(t + b + 4, b)
            return carry

        lax.fori_loop(1, NT_PER_W // 4 - 1, ring_body, 0)

        t = t0w + NT_PER_W - 4
        for b in range(4):
            wait_read(b)
            wait_owrite(b)
            transpose_tile(b)
            fire_owrite(t + b, b)
        for b in range(4):
            wait_owrite(b)

        @pl.when(wid < NT_EXTRA)
        def _extra():
            te = NW * NT_PER_W + wid
            fire_read(te, 0)
            wait_read(0)
            transpose_tile(0)
            fire_owrite(te, 0)
            wait_owrite(0)

    return k(embT)


def _acosh_body(d_ref, o_ref):
    d = d_ref[...]
    o_ref[...] = -jnp.log(d + jnp.sqrt((d - 1.0) * (d + 1.0)))


def _tc_neg_acosh(d):
    d2 = d.reshape(N // 128, 128)
    out = pl.pallas_call(
        _acosh_body,
        out_shape=jax.ShapeDtypeStruct((N // 128, 128), jnp.float32),
    )(d2)
    return out


def kernel(x, y, embedding):
    xf = x.reshape(N)
    yf = y.reshape(N)
    t128 = _sc_pack_table(embedding.T)
    tail = embedding[NT_FULL * 128:, :].reshape(TAIL_COLS // 8, 128)
    t128 = lax.dynamic_update_slice(t128, tail, (NT_FULL * 16, 0))
    d = _sc_distance(xf, yf, t128.reshape(N_TBL, D))
    return _tc_neg_acosh(d).reshape(x.shape)
